# baseline (speedup 1.0000x reference)
import functools

import jax
import jax.numpy as jnp
from jax import lax
from jax.experimental import pallas as pl
from jax.experimental.pallas import tpu as pltpu
from jax.experimental.pallas import tpu_sc as plsc

N = 1000000
DIM = 32
B = 16384
NC = 2
NS = 16
NW = NC * NS
BPW = B // NW
CHUNK = 128
KCH = BPW // CHUNK
GROUPS = BPW // 16

_LN2 = 0.6931471805599453


def _log_sigmoid(x):
    e = jnp.exp(-x)
    y = 1.0 / (1.0 + e) + 1e-20
    i = lax.bitcast_convert_type(y, jnp.int32)
    ix = i + jnp.int32(0x3F800000 - 0x3F3504F3)
    k = lax.shift_right_arithmetic(ix, jnp.int32(23)) - jnp.int32(127)
    mb = lax.bitwise_and(ix, jnp.int32(0x007FFFFF)) + jnp.int32(0x3F3504F3)
    m = lax.bitcast_convert_type(mb, jnp.float32)
    f = m - 1.0
    s = f / (2.0 + f)
    z = s * s
    w = z * z
    t1 = w * (0.40000972152 + w * 0.24279078841)
    t2 = z * (0.66666662693 + w * 0.28498786688)
    r = t2 + t1
    hfsq = 0.5 * f * f
    kf = k.astype(jnp.float32)
    return kf * _LN2 + (f - (hfsq - s * (hfsq + r)))


@functools.lru_cache(maxsize=1)
def _build_sc_kernel():
    mesh = plsc.VectorSubcoreMesh(core_axis_name="c", subcore_axis_name="s")

    @functools.partial(
        pl.kernel,
        mesh=mesh,
        compiler_params=pltpu.CompilerParams(
            needs_layout_passes=False, use_tc_tiling_on_sc=False),
        out_type=jax.ShapeDtypeStruct((B,), jnp.float32),
        scratch_types=[
            pltpu.VMEM((KCH, CHUNK), jnp.int32),
            pltpu.VMEM((KCH, CHUNK), jnp.int32),
            pltpu.VMEM((BPW, DIM), jnp.float32),
            pltpu.VMEM((BPW, DIM), jnp.float32),
            pltpu.VMEM((BPW,), jnp.float32),
            pltpu.SemaphoreType.DMA,
            pltpu.SemaphoreType.DMA,
        ],
    )
    def sc_kernel(idx1_hbm, idx2_hbm, t1_hbm, t2_hbm, out_hbm,
                  idx1_v, idx2_v, rows1, rows2, out_v, sem1, sem2):
        wid = lax.axis_index("s") * NC + lax.axis_index("c")
        base = wid * BPW

        pltpu.sync_copy(idx1_hbm.at[wid], idx1_v)
        pltpu.sync_copy(idx2_hbm.at[wid], idx2_v)

        copies = []
        for j in range(KCH):
            dst1 = rows1.at[pl.ds(j * CHUNK, CHUNK)]
            dst2 = rows2.at[pl.ds(j * CHUNK, CHUNK)]
            copies.append(pltpu.async_copy(t1_hbm.at[idx1_v.at[j]], dst1, sem1))
            copies.append(pltpu.async_copy(t2_hbm.at[idx2_v.at[j]], dst2, sem2))
        for c in copies:
            c.wait()

        lane = lax.iota(jnp.int32, 16)

        def group_body(g, carry):
            row_idx = lane + g * 16
            acc = jnp.zeros((16,), jnp.float32)
            for c in range(16):
                col_a = jnp.full((16,), c, jnp.int32)
                col_b = jnp.full((16,), c + 16, jnp.int32)
                acc = acc + plsc.load_gather(rows1, [row_idx, col_a]) * \
                    plsc.load_gather(rows2, [row_idx, col_a])
                acc = acc + plsc.load_gather(rows1, [row_idx, col_b]) * \
                    plsc.load_gather(rows2, [row_idx, col_b])
            out_v[pl.ds(g * 16, 16)] = _log_sigmoid(acc)
            return carry

        lax.fori_loop(0, GROUPS, group_body, 0)
        pltpu.sync_copy(out_v, out_hbm.at[pl.ds(base, BPW)])

    return sc_kernel


def kernel(idx1, idx2, emb1, emb2, embs1, embs2):
    del emb1, emb2
    i1 = idx1.astype(jnp.int32).reshape(NW, KCH, CHUNK)
    i2 = idx2.astype(jnp.int32).reshape(NW, KCH, CHUNK)
    return _build_sc_kernel()(i1, i2, embs1, embs2)
